# baseline (speedup 1.0000x reference)
import functools

import jax
import jax.numpy as jnp
from jax import lax
from jax.experimental import pallas as pl
from jax.experimental.pallas import tpu as pltpu
from jax.experimental.pallas import tpu_sc as plsc

T, D_IN, D_OUT, E, RANK, TOP_K = 4096, 1024, 1024, 8, 16, 2
TILE = 1024

NC, NS, L = 2, 16, 16
NW = NC * NS
TPW = T // NW


def _tc1(x_ref, wb_ref, bb_ref, wr_ref, af_ref, ob_ref, lt_ref, h_ref, wb_s):
    @pl.when(pl.program_id(0) == 0)
    def _prep():
        wb_s[...] = wb_ref[...].astype(jnp.bfloat16)

    x = x_ref[...]
    xh = x.astype(jnp.bfloat16)
    lt_ref[...] = jax.lax.dot_general(
        wr_ref[...], x, (((1,), (1,)), ((), ())),
        preferred_element_type=jnp.float32)
    h_ref[...] = jax.lax.dot_general(
        xh, af_ref[...], (((1,), (1,)), ((), ())),
        preferred_element_type=jnp.float32)
    base = jax.lax.dot_general(
        xh, wb_s[...], (((1,), (1,)), ((), ())),
        preferred_element_type=jnp.float32)
    ob_ref[...] = base + bb_ref[...]


def _sc_gate(lt_hbm, g_hbm, l_v, g_v):
    wid = lax.axis_index("s") * NC + lax.axis_index("c")
    base = wid * TPW
    pltpu.sync_copy(lt_hbm.at[:, pl.ds(base, TPW)], l_v)
    for c in range(TPW // L):
        sl = pl.ds(c * L, L)
        ls = [l_v[e, sl] for e in range(E)]
        m = ls[0]
        for e in range(1, E):
            m = jnp.maximum(m, ls[e])
        egs = [jnp.exp(l - m) for l in ls]
        s_all = egs[0]
        for e in range(1, E):
            s_all = s_all + egs[e]
        i1 = jnp.where(ls[0] == m, 0, E)
        for e in range(1, E):
            i1 = jnp.minimum(i1, jnp.where(ls[e] == m, e, E))
        NEG = jnp.float32(-3e38)
        m2 = jnp.where(i1 == 0, NEG, ls[0])
        for e in range(1, E):
            m2 = jnp.maximum(m2, jnp.where(i1 == e, NEG, ls[e]))
        i2 = jnp.where((ls[0] == m2) & (i1 != 0), 0, E)
        for e in range(1, E):
            i2 = jnp.minimum(i2, jnp.where((ls[e] == m2) & (i1 != e), e, E))
        egms = []
        s_top = None
        for e in range(E):
            sel = (i1 == e) | (i2 == e)
            egm = jnp.where(sel, egs[e], 0.0)
            egms.append(egm)
            s_top = egm if s_top is None else s_top + egm
        inv = 1.0 / (s_top + 1e-6 * s_all)
        for e in range(E):
            g_v[e, sl] = egms[e] * inv
    pltpu.sync_copy(g_v, g_hbm.at[:, pl.ds(base, TPW)])


def _tc2(ob_ref, h_ref, g_ref, bf_ref, out_ref):
    gate_nT = g_ref[...].astype(jnp.bfloat16)
    re = jax.lax.broadcasted_iota(jnp.int32, (E, E * RANK), 0)
    rc = jax.lax.broadcasted_iota(jnp.int32, (E, E * RANK), 1)
    rep = (rc // RANK == re).astype(jnp.bfloat16)
    gate_rep = jax.lax.dot_general(
        gate_nT, rep, (((0,), (0,)), ((), ())),
        preferred_element_type=jnp.float32)
    hw = (h_ref[...] * gate_rep).astype(jnp.bfloat16)
    lora = jax.lax.dot_general(
        hw, bf_ref[...], (((1,), (0,)), ((), ())),
        preferred_element_type=jnp.float32)
    out_ref[...] = ob_ref[...] + lora


def kernel(x, W_base, b_base, W_router, A, B):
    af = A.reshape(E * RANK, D_IN).astype(jnp.bfloat16)
    bf = jnp.transpose(B, (0, 2, 1)).reshape(E * RANK, D_OUT).astype(jnp.bfloat16)
    bb = b_base.reshape(1, D_OUT)

    grid = (T // TILE,)
    ob, lt, h = pl.pallas_call(
        _tc1,
        grid=grid,
        in_specs=[
            pl.BlockSpec((TILE, D_IN), lambda i: (i, 0)),
            pl.BlockSpec((D_OUT, D_IN), lambda i: (0, 0)),
            pl.BlockSpec((1, D_OUT), lambda i: (0, 0)),
            pl.BlockSpec((E, D_IN), lambda i: (0, 0)),
            pl.BlockSpec((E * RANK, D_IN), lambda i: (0, 0)),
        ],
        out_specs=[
            pl.BlockSpec((TILE, D_OUT), lambda i: (i, 0)),
            pl.BlockSpec((E, TILE), lambda i: (0, i)),
            pl.BlockSpec((TILE, E * RANK), lambda i: (i, 0)),
        ],
        out_shape=[
            jax.ShapeDtypeStruct((T, D_OUT), jnp.float32),
            jax.ShapeDtypeStruct((E, T), jnp.float32),
            jax.ShapeDtypeStruct((T, E * RANK), jnp.float32),
        ],
        scratch_shapes=[pltpu.VMEM((D_OUT, D_IN), jnp.bfloat16)],
        compiler_params=pltpu.CompilerParams(
            dimension_semantics=("arbitrary",),
        ),
    )(x, W_base, bb, W_router, af)

    sc_gate = functools.partial(
        pl.kernel,
        mesh=plsc.VectorSubcoreMesh(core_axis_name="c", subcore_axis_name="s"),
        out_type=jax.ShapeDtypeStruct((E, T), jnp.float32),
        scratch_types=[
            pltpu.VMEM((E, TPW), jnp.float32),
            pltpu.VMEM((E, TPW), jnp.float32),
        ],
    )(_sc_gate)
    g = sc_gate(lt)

    out = pl.pallas_call(
        _tc2,
        grid=grid,
        in_specs=[
            pl.BlockSpec((TILE, D_OUT), lambda i: (i, 0)),
            pl.BlockSpec((TILE, E * RANK), lambda i: (i, 0)),
            pl.BlockSpec((E, TILE), lambda i: (0, i)),
            pl.BlockSpec((E * RANK, D_OUT), lambda i: (0, 0)),
        ],
        out_specs=pl.BlockSpec((TILE, D_OUT), lambda i: (i, 0)),
        out_shape=jax.ShapeDtypeStruct((T, D_OUT), jnp.float32),
        compiler_params=pltpu.CompilerParams(
            dimension_semantics=("arbitrary",),
        ),
    )(ob, h, g, bf)
    return out


# speedup vs baseline: 2.0108x; 2.0108x over previous
import jax
import jax.numpy as jnp
from jax.experimental import pallas as pl
from jax.experimental.pallas import tpu as pltpu

T, D_IN, D_OUT, E, RANK, TOP_K = 4096, 1024, 1024, 8, 16, 2
TILE = 1024


def _fused_kernel(x_ref, wb_ref, bb_ref, wr_ref, af_ref, bf_ref, out_ref,
                  wb_s):
    @pl.when(pl.program_id(0) == 0)
    def _prep():
        wb_s[...] = wb_ref[...].astype(jnp.bfloat16)

    x = x_ref[...]
    xh = x.astype(jnp.bfloat16)

    lT = jax.lax.dot_general(
        wr_ref[...], x, (((1,), (1,)), ((), ())),
        preferred_element_type=jnp.float32)
    m = jnp.max(lT, axis=0, keepdims=True)
    eg = jnp.exp(lT - m)
    s_all = jnp.sum(eg, axis=0, keepdims=True)

    eidx = jax.lax.broadcasted_iota(jnp.int32, (E, TILE), 0)
    i1 = jnp.min(jnp.where(lT == m, eidx, E), axis=0, keepdims=True)
    mask1 = eidx == i1
    l2 = jnp.where(mask1, float("-inf"), lT)
    m2 = jnp.max(l2, axis=0, keepdims=True)
    i2 = jnp.min(jnp.where(l2 == m2, eidx, E), axis=0, keepdims=True)
    mask = mask1 | (eidx == i2)

    egm = jnp.where(mask, eg, 0.0)
    s_top = jnp.sum(egm, axis=0, keepdims=True)
    gate_nT = (egm / (s_top + 1e-6 * s_all)).astype(jnp.bfloat16)

    re = jax.lax.broadcasted_iota(jnp.int32, (E, E * RANK), 0)
    rc = jax.lax.broadcasted_iota(jnp.int32, (E, E * RANK), 1)
    rep = (rc // RANK == re).astype(jnp.bfloat16)
    gate_rep = jax.lax.dot_general(
        gate_nT, rep, (((0,), (0,)), ((), ())),
        preferred_element_type=jnp.float32)

    h = jax.lax.dot_general(
        xh, af_ref[...], (((1,), (1,)), ((), ())),
        preferred_element_type=jnp.float32)
    hw = (h * gate_rep).astype(jnp.bfloat16)

    CH = D_OUT // 4
    for c in range(4):
        sl = pl.ds(c * CH, CH)
        base_c = jax.lax.dot_general(
            xh, wb_s[sl, :], (((1,), (1,)), ((), ())),
            preferred_element_type=jnp.float32)
        lora_c = jax.lax.dot_general(
            hw, bf_ref[:, sl], (((1,), (0,)), ((), ())),
            preferred_element_type=jnp.float32)
        out_ref[:, sl] = base_c + bb_ref[:, sl] + lora_c


def kernel(x, W_base, b_base, W_router, A, B):
    af = A.reshape(E * RANK, D_IN).astype(jnp.bfloat16)
    bf = jnp.transpose(B, (0, 2, 1)).reshape(E * RANK, D_OUT).astype(jnp.bfloat16)
    bb = b_base.reshape(1, D_OUT)

    grid = (T // TILE,)
    return pl.pallas_call(
        _fused_kernel,
        grid=grid,
        in_specs=[
            pl.BlockSpec((TILE, D_IN), lambda i: (i, 0)),
            pl.BlockSpec((D_OUT, D_IN), lambda i: (0, 0)),
            pl.BlockSpec((1, D_OUT), lambda i: (0, 0)),
            pl.BlockSpec((E, D_IN), lambda i: (0, 0)),
            pl.BlockSpec((E * RANK, D_IN), lambda i: (0, 0)),
            pl.BlockSpec((E * RANK, D_OUT), lambda i: (0, 0)),
        ],
        out_specs=pl.BlockSpec((TILE, D_OUT), lambda i: (i, 0)),
        out_shape=jax.ShapeDtypeStruct((T, D_OUT), jnp.float32),
        scratch_shapes=[pltpu.VMEM((D_OUT, D_IN), jnp.bfloat16)],
        compiler_params=pltpu.CompilerParams(
            dimension_semantics=("arbitrary",),
        ),
    )(x, W_base, bb, W_router, af, bf)
